# Initial kernel scaffold; baseline (speedup 1.0000x reference)
import functools

import jax
import jax.numpy as jnp
from jax import lax
from jax.experimental import pallas as pl
from jax.experimental.pallas import tpu as pltpu
from jax.experimental.pallas import tpu_sc as plsc

N = 10000
DF = 128
DH = 64
NUM_ITER = 10
ALPHA = 0.1
NC, NS, L = 2, 16, 16
NW = NC * NS
CHUNK = 128
ACC_ROWS = 10240
RPS = ACC_ROWS // NS
BLK = 1000


def _sc_mesh():
    return plsc.VectorSubcoreMesh(core_axis_name="c", subcore_axis_name="s")


def _mlp_body(x_ref, w0_ref, b0_ref, w1_ref, b1_ref, o_ref):
    h = jnp.maximum(
        jnp.dot(x_ref[...], w0_ref[...], preferred_element_type=jnp.float32)
        + b0_ref[...], 0.0)
    o_ref[...] = (
        jnp.dot(h, w1_ref[...], preferred_element_type=jnp.float32)
        + b1_ref[...])


def _mlp(x, W0, b0, W1, b1):
    return pl.pallas_call(
        _mlp_body,
        grid=(N // BLK,),
        in_specs=[
            pl.BlockSpec((BLK, DF), lambda i: (i, 0)),
            pl.BlockSpec((DF, DF), lambda i: (0, 0)),
            pl.BlockSpec((1, DF), lambda i: (0, 0)),
            pl.BlockSpec((DF, DH), lambda i: (0, 0)),
            pl.BlockSpec((1, DH), lambda i: (0, 0)),
        ],
        out_specs=pl.BlockSpec((BLK, DH), lambda i: (i, 0)),
        out_shape=jax.ShapeDtypeStruct((N, DH), jnp.float32),
    )(x, W0, b0.reshape(1, DF), W1, b1.reshape(1, DH))


def _make_deg_kernel(cpw):
    @functools.partial(
        pl.kernel,
        mesh=_sc_mesh(),
        out_type=jax.ShapeDtypeStruct((NC, ACC_ROWS, L), jnp.float32),
        scratch_types=[
            pltpu.VMEM((cpw, CHUNK), jnp.int32),
            pltpu.VMEM((CHUNK, L), jnp.float32),
            pltpu.VMEM_SHARED((ACC_ROWS, L), jnp.float32),
            pltpu.SemaphoreType.DMA,
        ],
    )
    def k(rowi_hbm, ones_hbm, zeros_hbm, out_hbm, idx_v, ones_v, acc_s, sem):
        cid = lax.axis_index("c")
        sid = lax.axis_index("s")
        wid = sid * NC + cid
        pltpu.sync_copy(rowi_hbm.at[wid], idx_v)
        pltpu.sync_copy(ones_hbm, ones_v)
        pltpu.sync_copy(zeros_hbm.at[pl.ds(sid * RPS, RPS)],
                        acc_s.at[pl.ds(sid * RPS, RPS)])
        plsc.subcore_barrier()

        @pl.loop(0, cpw)
        def _(c):
            pltpu.sync_copy(ones_v, acc_s.at[idx_v.at[c]], add=True)

        plsc.subcore_barrier()
        pltpu.sync_copy(acc_s.at[pl.ds(sid * RPS, RPS)],
                        out_hbm.at[cid, pl.ds(sid * RPS, RPS)])

    return k


def _make_prop_kernel(cpw):
    @functools.partial(
        pl.kernel,
        mesh=_sc_mesh(),
        out_type=jax.ShapeDtypeStruct((NC, ACC_ROWS, DH), jnp.float32),
        scratch_types=[
            pltpu.VMEM((cpw, CHUNK), jnp.int32),
            pltpu.VMEM((cpw, CHUNK), jnp.int32),
            pltpu.VMEM((CHUNK, DH), jnp.float32),
            pltpu.VMEM_SHARED((ACC_ROWS, DH), jnp.float32),
            pltpu.SemaphoreType.DMA,
        ],
    )
    def k(g_hbm, rowi_hbm, coli_hbm, zeros_hbm, out_hbm,
          ridx_v, cidx_v, buf, acc_s, sem):
        cid = lax.axis_index("c")
        sid = lax.axis_index("s")
        wid = sid * NC + cid
        pltpu.sync_copy(rowi_hbm.at[wid], ridx_v)
        pltpu.sync_copy(coli_hbm.at[wid], cidx_v)
        pltpu.sync_copy(zeros_hbm.at[pl.ds(sid * RPS, RPS)],
                        acc_s.at[pl.ds(sid * RPS, RPS)])
        plsc.subcore_barrier()

        @pl.loop(0, cpw)
        def _(c):
            pltpu.async_copy(g_hbm.at[cidx_v.at[c]], buf, sem).wait()
            pltpu.sync_copy(buf, acc_s.at[ridx_v.at[c]], add=True)

        plsc.subcore_barrier()
        pltpu.sync_copy(acc_s.at[pl.ds(sid * RPS, RPS)],
                        out_hbm.at[cid, pl.ds(sid * RPS, RPS)])

    return k


def _coeff_body(degp_ref, h0_ref, g0_ref, am_ref, bm_ref, af_ref, bf_ref):
    deg = degp_ref[0, :, 0:1] + degp_ref[1, :, 0:1] + 1.0
    dinv = lax.rsqrt(deg)
    g0_ref[...] = dinv * h0_ref[...]
    am_ref[...] = (1.0 - ALPHA) * dinv * dinv
    bm_ref[...] = ALPHA * dinv
    af_ref[...] = (1.0 - ALPHA) * dinv
    bf_ref[...] = jnp.full_like(dinv, ALPHA)


def _coeff(degp, h0):
    one_col = jax.ShapeDtypeStruct((N, 1), jnp.float32)
    return pl.pallas_call(
        _coeff_body,
        grid=(N // BLK,),
        in_specs=[
            pl.BlockSpec((NC, BLK, L), lambda i: (0, i, 0)),
            pl.BlockSpec((BLK, DH), lambda i: (i, 0)),
        ],
        out_specs=[pl.BlockSpec((BLK, DH), lambda i: (i, 0))]
        + [pl.BlockSpec((BLK, 1), lambda i: (i, 0))] * 4,
        out_shape=[jax.ShapeDtypeStruct((N, DH), jnp.float32)] + [one_col] * 4,
    )(degp, h0)


def _combine_body(p_ref, g_ref, h0_ref, a_ref, b_ref, o_ref):
    s = p_ref[0] + p_ref[1] + g_ref[...]
    o_ref[...] = a_ref[...] * s + b_ref[...] * h0_ref[...]


def _combine(p, g, h0, a, b):
    return pl.pallas_call(
        _combine_body,
        grid=(N // BLK,),
        in_specs=[
            pl.BlockSpec((NC, BLK, DH), lambda i: (0, i, 0)),
            pl.BlockSpec((BLK, DH), lambda i: (i, 0)),
            pl.BlockSpec((BLK, DH), lambda i: (i, 0)),
            pl.BlockSpec((BLK, 1), lambda i: (i, 0)),
            pl.BlockSpec((BLK, 1), lambda i: (i, 0)),
        ],
        out_specs=pl.BlockSpec((BLK, DH), lambda i: (i, 0)),
        out_shape=jax.ShapeDtypeStruct((N, DH), jnp.float32),
    )(p, g, h0, a, b)


def kernel(x, edge_index, W0, b0, W1, b1):
    ei = edge_index.astype(jnp.int32)
    row, col = ei[0], ei[1]
    ne = row.shape[0]
    cpw = -(-ne // (NW * CHUNK))
    cpw += cpw % 2
    ne_pad = NW * cpw * CHUNK
    row_p = jnp.concatenate([row, jnp.full((ne_pad - ne,), N, jnp.int32)])
    col_p = jnp.concatenate([col, jnp.zeros((ne_pad - ne,), jnp.int32)])
    rowi = row_p.reshape(NW, cpw, CHUNK)
    coli = col_p.reshape(NW, cpw, CHUNK)
    zeros_d = jnp.zeros((ACC_ROWS, DH), jnp.float32)
    zeros_l = jnp.zeros((ACC_ROWS, L), jnp.float32)
    ones_l = jnp.ones((CHUNK, L), jnp.float32)

    h0 = _mlp(x, W0, b0, W1, b1)
    degp = _make_deg_kernel(cpw)(rowi, ones_l, zeros_l)
    g, am, bm, af, bf = _coeff(degp, h0)
    prop = _make_prop_kernel(cpw)
    for t in range(NUM_ITER):
        p = prop(g, rowi, coli, zeros_d)
        if t < NUM_ITER - 1:
            g = _combine(p, g, h0, am, bm)
        else:
            g = _combine(p, g, h0, af, bf)
    return g


# speedup vs baseline: 8.7843x; 8.7843x over previous
import functools

import jax
import jax.numpy as jnp
from jax import lax
from jax.experimental import pallas as pl
from jax.experimental.pallas import tpu as pltpu
from jax.experimental.pallas import tpu_sc as plsc

N = 10000
DF = 128
DH = 64
NUM_ITER = 10
ALPHA = 0.1
NC, NS, L = 2, 16, 16
NW = NC * NS
CHUNK = 128
ACC_ROWS = 10240
RPS = ACC_ROWS // NS
BLK = 1000


def _sc_mesh():
    return plsc.VectorSubcoreMesh(core_axis_name="c", subcore_axis_name="s")


_SC_PARAMS = pltpu.CompilerParams(use_tc_tiling_on_sc=False)


def _mlp_body(x_ref, w0_ref, b0_ref, w1_ref, b1_ref, o_ref):
    h = jnp.maximum(
        jnp.dot(x_ref[...], w0_ref[...], preferred_element_type=jnp.float32)
        + b0_ref[...], 0.0)
    o_ref[...] = (
        jnp.dot(h, w1_ref[...], preferred_element_type=jnp.float32)
        + b1_ref[...])


def _mlp(x, W0, b0, W1, b1):
    return pl.pallas_call(
        _mlp_body,
        grid=(N // BLK,),
        in_specs=[
            pl.BlockSpec((BLK, DF), lambda i: (i, 0)),
            pl.BlockSpec((DF, DF), lambda i: (0, 0)),
            pl.BlockSpec((1, DF), lambda i: (0, 0)),
            pl.BlockSpec((DF, DH), lambda i: (0, 0)),
            pl.BlockSpec((1, DH), lambda i: (0, 0)),
        ],
        out_specs=pl.BlockSpec((BLK, DH), lambda i: (i, 0)),
        out_shape=jax.ShapeDtypeStruct((N, DH), jnp.float32),
    )(x, W0, b0.reshape(1, DF), W1, b1.reshape(1, DH))


def _make_deg_kernel(cpw):
    @functools.partial(
        pl.kernel,
        mesh=_sc_mesh(),
        out_type=jax.ShapeDtypeStruct((NC, ACC_ROWS, L), jnp.float32),
        scratch_types=[
            pltpu.VMEM((cpw, CHUNK), jnp.int32),
            pltpu.VMEM((CHUNK, L), jnp.float32),
            pltpu.VMEM_SHARED((ACC_ROWS, L), jnp.float32),
            pltpu.SemaphoreType.DMA,
        ],
        compiler_params=_SC_PARAMS,
    )
    def k(rowi_hbm, ones_hbm, zeros_hbm, out_hbm, idx_v, ones_v, acc_s, sem):
        cid = lax.axis_index("c")
        sid = lax.axis_index("s")
        wid = sid * NC + cid
        pltpu.sync_copy(rowi_hbm.at[wid], idx_v)
        pltpu.sync_copy(ones_hbm, ones_v)
        pltpu.sync_copy(zeros_hbm.at[pl.ds(sid * RPS, RPS)],
                        acc_s.at[pl.ds(sid * RPS, RPS)])
        plsc.subcore_barrier()

        @pl.loop(0, cpw)
        def _(c):
            pltpu.sync_copy(ones_v, acc_s.at[idx_v.at[c]], add=True)

        plsc.subcore_barrier()
        pltpu.sync_copy(acc_s.at[pl.ds(sid * RPS, RPS)],
                        out_hbm.at[cid, pl.ds(sid * RPS, RPS)])

    return k


def _make_prop_kernel(cpw):
    @functools.partial(
        pl.kernel,
        mesh=_sc_mesh(),
        out_type=jax.ShapeDtypeStruct((NC, ACC_ROWS, DH), jnp.float32),
        scratch_types=[
            pltpu.VMEM((cpw, CHUNK), jnp.int32),
            pltpu.VMEM((cpw, CHUNK), jnp.int32),
            pltpu.VMEM((CHUNK, DH), jnp.float32),
            pltpu.VMEM_SHARED((ACC_ROWS, DH), jnp.float32),
            pltpu.SemaphoreType.DMA,
        ],
        compiler_params=_SC_PARAMS,
    )
    def k(g_hbm, rowi_hbm, coli_hbm, zeros_hbm, out_hbm,
          ridx_v, cidx_v, buf, acc_s, sem):
        cid = lax.axis_index("c")
        sid = lax.axis_index("s")
        wid = sid * NC + cid
        pltpu.sync_copy(rowi_hbm.at[wid], ridx_v)
        pltpu.sync_copy(coli_hbm.at[wid], cidx_v)
        pltpu.sync_copy(zeros_hbm.at[pl.ds(sid * RPS, RPS)],
                        acc_s.at[pl.ds(sid * RPS, RPS)])
        plsc.subcore_barrier()

        @pl.loop(0, cpw)
        def _(c):
            pltpu.async_copy(g_hbm.at[cidx_v.at[c]], buf, sem).wait()
            pltpu.sync_copy(buf, acc_s.at[ridx_v.at[c]], add=True)

        plsc.subcore_barrier()
        pltpu.sync_copy(acc_s.at[pl.ds(sid * RPS, RPS)],
                        out_hbm.at[cid, pl.ds(sid * RPS, RPS)])

    return k


def _coeff_body(degp_ref, h0_ref, g0_ref, am_ref, bm_ref, af_ref, bf_ref):
    deg = degp_ref[0, :, 0:1] + degp_ref[1, :, 0:1] + 1.0
    dinv = lax.rsqrt(deg)
    g0_ref[...] = dinv * h0_ref[...]
    am_ref[...] = (1.0 - ALPHA) * dinv * dinv
    bm_ref[...] = ALPHA * dinv
    af_ref[...] = (1.0 - ALPHA) * dinv
    bf_ref[...] = jnp.full_like(dinv, ALPHA)


def _coeff(degp, h0):
    one_col = jax.ShapeDtypeStruct((N, 1), jnp.float32)
    return pl.pallas_call(
        _coeff_body,
        grid=(N // BLK,),
        in_specs=[
            pl.BlockSpec((NC, BLK, L), lambda i: (0, i, 0)),
            pl.BlockSpec((BLK, DH), lambda i: (i, 0)),
        ],
        out_specs=[pl.BlockSpec((BLK, DH), lambda i: (i, 0))]
        + [pl.BlockSpec((BLK, 1), lambda i: (i, 0))] * 4,
        out_shape=[jax.ShapeDtypeStruct((N, DH), jnp.float32)] + [one_col] * 4,
    )(degp, h0)


def _combine_body(p_ref, g_ref, h0_ref, a_ref, b_ref, o_ref):
    s = p_ref[0] + p_ref[1] + g_ref[...]
    o_ref[...] = a_ref[...] * s + b_ref[...] * h0_ref[...]


def _combine(p, g, h0, a, b):
    return pl.pallas_call(
        _combine_body,
        grid=(N // BLK,),
        in_specs=[
            pl.BlockSpec((NC, BLK, DH), lambda i: (0, i, 0)),
            pl.BlockSpec((BLK, DH), lambda i: (i, 0)),
            pl.BlockSpec((BLK, DH), lambda i: (i, 0)),
            pl.BlockSpec((BLK, 1), lambda i: (i, 0)),
            pl.BlockSpec((BLK, 1), lambda i: (i, 0)),
        ],
        out_specs=pl.BlockSpec((BLK, DH), lambda i: (i, 0)),
        out_shape=jax.ShapeDtypeStruct((N, DH), jnp.float32),
    )(p, g, h0, a, b)


def kernel(x, edge_index, W0, b0, W1, b1):
    ei = edge_index.astype(jnp.int32)
    row, col = ei[0], ei[1]
    ne = row.shape[0]
    cpw = -(-ne // (NW * CHUNK))
    cpw += cpw % 2
    ne_pad = NW * cpw * CHUNK
    row_p = jnp.concatenate([row, jnp.full((ne_pad - ne,), N, jnp.int32)])
    col_p = jnp.concatenate([col, jnp.zeros((ne_pad - ne,), jnp.int32)])
    rowi = row_p.reshape(NW, cpw, CHUNK)
    coli = col_p.reshape(NW, cpw, CHUNK)
    zeros_d = jnp.zeros((ACC_ROWS, DH), jnp.float32)
    zeros_l = jnp.zeros((ACC_ROWS, L), jnp.float32)
    ones_l = jnp.ones((CHUNK, L), jnp.float32)

    h0 = _mlp(x, W0, b0, W1, b1)
    degp = _make_deg_kernel(cpw)(rowi, ones_l, zeros_l)
    g, am, bm, af, bf = _coeff(degp, h0)
    prop = _make_prop_kernel(cpw)
    for t in range(NUM_ITER):
        p = prop(g, rowi, coli, zeros_d)
        if t < NUM_ITER - 1:
            g = _combine(p, g, h0, am, bm)
        else:
            g = _combine(p, g, h0, af, bf)
    return g


# speedup vs baseline: 10.3582x; 1.1792x over previous
import functools

import jax
import jax.numpy as jnp
from jax import lax
from jax.experimental import pallas as pl
from jax.experimental.pallas import tpu as pltpu
from jax.experimental.pallas import tpu_sc as plsc

N = 10000
DF = 128
DH = 64
NUM_ITER = 10
ALPHA = 0.1
NC, NS, L = 2, 16, 16
NW = NC * NS
CHUNK = 128
ACC_ROWS = 10240
RPS = ACC_ROWS // NS
BLK = 1000


def _sc_mesh():
    return plsc.VectorSubcoreMesh(core_axis_name="c", subcore_axis_name="s")


_SC_PARAMS = pltpu.CompilerParams(use_tc_tiling_on_sc=False)


def _mlp_body(x_ref, w0_ref, b0_ref, w1_ref, b1_ref, o_ref):
    h = jnp.maximum(
        jnp.dot(x_ref[...], w0_ref[...], preferred_element_type=jnp.float32)
        + b0_ref[...], 0.0)
    o_ref[...] = (
        jnp.dot(h, w1_ref[...], preferred_element_type=jnp.float32)
        + b1_ref[...])


def _mlp(x, W0, b0, W1, b1):
    return pl.pallas_call(
        _mlp_body,
        grid=(N // BLK,),
        in_specs=[
            pl.BlockSpec((BLK, DF), lambda i: (i, 0)),
            pl.BlockSpec((DF, DF), lambda i: (0, 0)),
            pl.BlockSpec((1, DF), lambda i: (0, 0)),
            pl.BlockSpec((DF, DH), lambda i: (0, 0)),
            pl.BlockSpec((1, DH), lambda i: (0, 0)),
        ],
        out_specs=pl.BlockSpec((BLK, DH), lambda i: (i, 0)),
        out_shape=jax.ShapeDtypeStruct((N, DH), jnp.float32),
    )(x, W0, b0.reshape(1, DF), W1, b1.reshape(1, DH))


def _make_deg_kernel(cpw):
    @functools.partial(
        pl.kernel,
        mesh=_sc_mesh(),
        out_type=jax.ShapeDtypeStruct((NC, ACC_ROWS, L), jnp.float32),
        scratch_types=[
            pltpu.VMEM((cpw, CHUNK), jnp.int32),
            pltpu.VMEM((CHUNK, L), jnp.float32),
            pltpu.VMEM_SHARED((ACC_ROWS, L), jnp.float32),
            pltpu.SemaphoreType.DMA,
        ],
        compiler_params=_SC_PARAMS,
    )
    def k(rowi_hbm, ones_hbm, zeros_hbm, out_hbm, idx_v, ones_v, acc_s, sem):
        cid = lax.axis_index("c")
        sid = lax.axis_index("s")
        wid = sid * NC + cid
        pltpu.sync_copy(rowi_hbm.at[wid], idx_v)
        pltpu.sync_copy(ones_hbm, ones_v)
        pltpu.sync_copy(zeros_hbm.at[pl.ds(sid * RPS, RPS)],
                        acc_s.at[pl.ds(sid * RPS, RPS)])
        plsc.subcore_barrier()

        @pl.loop(0, cpw)
        def _(c):
            pltpu.sync_copy(ones_v, acc_s.at[idx_v.at[c]], add=True)

        plsc.subcore_barrier()
        pltpu.sync_copy(acc_s.at[pl.ds(sid * RPS, RPS)],
                        out_hbm.at[cid, pl.ds(sid * RPS, RPS)])

    return k


NBUF = 8


def _make_prop_kernel(cpw):
    assert cpw % NBUF == 0 and cpw // NBUF >= 2

    @functools.partial(
        pl.kernel,
        mesh=_sc_mesh(),
        out_type=jax.ShapeDtypeStruct((NC, ACC_ROWS, DH), jnp.float32),
        scratch_types=[
            pltpu.VMEM((cpw, CHUNK), jnp.int32),
            pltpu.VMEM((cpw, CHUNK), jnp.int32),
            [pltpu.VMEM((CHUNK, DH), jnp.float32)] * NBUF,
            [pltpu.SemaphoreType.DMA] * NBUF,
            [pltpu.SemaphoreType.DMA] * NBUF,
            pltpu.VMEM_SHARED((ACC_ROWS, DH), jnp.float32),
        ],
        compiler_params=_SC_PARAMS,
    )
    def k(g_hbm, rowi_hbm, coli_hbm, zeros_hbm, out_hbm,
          ridx_v, cidx_v, bufs, gsem, ssem, acc_s):
        cid = lax.axis_index("c")
        sid = lax.axis_index("s")
        wid = sid * NC + cid
        pltpu.sync_copy(rowi_hbm.at[wid], ridx_v)
        pltpu.sync_copy(coli_hbm.at[wid], cidx_v)
        pltpu.sync_copy(zeros_hbm.at[pl.ds(sid * RPS, RPS)],
                        acc_s.at[pl.ds(sid * RPS, RPS)])
        plsc.subcore_barrier()

        def gather(c, b):
            pltpu.async_copy(g_hbm.at[cidx_v.at[c]], bufs[b], gsem[b])

        def gather_wait(c, b):
            pltpu.make_async_copy(g_hbm.at[cidx_v.at[c]], bufs[b],
                                  gsem[b]).wait()

        def scatter(c, b):
            pltpu.async_copy(bufs[b], acc_s.at[ridx_v.at[c]], ssem[b],
                             add=True)

        def scatter_wait(c, b):
            pltpu.make_async_copy(bufs[b], acc_s.at[ridx_v.at[c]],
                                  ssem[b]).wait()

        for b in range(NBUF):
            gather(b, b)

        @pl.loop(0, cpw // NBUF - 1)
        def _(i):
            c0 = i * NBUF
            for b in range(NBUF):
                gather_wait(c0 + b, b)
                scatter(c0 + b, b)
            for b in range(NBUF):
                scatter_wait(c0 + b, b)
                gather(c0 + NBUF + b, b)

        c0 = cpw - NBUF
        for b in range(NBUF):
            gather_wait(c0 + b, b)
            scatter(c0 + b, b)
        for b in range(NBUF):
            scatter_wait(c0 + b, b)

        plsc.subcore_barrier()
        pltpu.sync_copy(acc_s.at[pl.ds(sid * RPS, RPS)],
                        out_hbm.at[cid, pl.ds(sid * RPS, RPS)])

    return k


def _coeff_body(degp_ref, h0_ref, g0_ref, am_ref, bm_ref, af_ref, bf_ref):
    deg = degp_ref[0, :, 0:1] + degp_ref[1, :, 0:1] + 1.0
    dinv = lax.rsqrt(deg)
    g0_ref[...] = dinv * h0_ref[...]
    am_ref[...] = (1.0 - ALPHA) * dinv * dinv
    bm_ref[...] = ALPHA * dinv
    af_ref[...] = (1.0 - ALPHA) * dinv
    bf_ref[...] = jnp.full_like(dinv, ALPHA)


def _coeff(degp, h0):
    one_col = jax.ShapeDtypeStruct((N, 1), jnp.float32)
    return pl.pallas_call(
        _coeff_body,
        grid=(N // BLK,),
        in_specs=[
            pl.BlockSpec((NC, BLK, L), lambda i: (0, i, 0)),
            pl.BlockSpec((BLK, DH), lambda i: (i, 0)),
        ],
        out_specs=[pl.BlockSpec((BLK, DH), lambda i: (i, 0))]
        + [pl.BlockSpec((BLK, 1), lambda i: (i, 0))] * 4,
        out_shape=[jax.ShapeDtypeStruct((N, DH), jnp.float32)] + [one_col] * 4,
    )(degp, h0)


def _combine_body(p_ref, g_ref, h0_ref, a_ref, b_ref, o_ref):
    s = p_ref[0] + p_ref[1] + g_ref[...]
    o_ref[...] = a_ref[...] * s + b_ref[...] * h0_ref[...]


def _combine(p, g, h0, a, b):
    return pl.pallas_call(
        _combine_body,
        grid=(N // BLK,),
        in_specs=[
            pl.BlockSpec((NC, BLK, DH), lambda i: (0, i, 0)),
            pl.BlockSpec((BLK, DH), lambda i: (i, 0)),
            pl.BlockSpec((BLK, DH), lambda i: (i, 0)),
            pl.BlockSpec((BLK, 1), lambda i: (i, 0)),
            pl.BlockSpec((BLK, 1), lambda i: (i, 0)),
        ],
        out_specs=pl.BlockSpec((BLK, DH), lambda i: (i, 0)),
        out_shape=jax.ShapeDtypeStruct((N, DH), jnp.float32),
    )(p, g, h0, a, b)


def kernel(x, edge_index, W0, b0, W1, b1):
    ei = edge_index.astype(jnp.int32)
    row, col = ei[0], ei[1]
    ne = row.shape[0]
    cpw = -(-ne // (NW * CHUNK))
    cpw = -(-cpw // NBUF) * NBUF
    ne_pad = NW * cpw * CHUNK
    row_p = jnp.concatenate([row, jnp.full((ne_pad - ne,), N, jnp.int32)])
    col_p = jnp.concatenate([col, jnp.zeros((ne_pad - ne,), jnp.int32)])
    rowi = row_p.reshape(NW, cpw, CHUNK)
    coli = col_p.reshape(NW, cpw, CHUNK)
    zeros_d = jnp.zeros((ACC_ROWS, DH), jnp.float32)
    zeros_l = jnp.zeros((ACC_ROWS, L), jnp.float32)
    ones_l = jnp.ones((CHUNK, L), jnp.float32)

    h0 = _mlp(x, W0, b0, W1, b1)
    degp = _make_deg_kernel(cpw)(rowi, ones_l, zeros_l)
    g, am, bm, af, bf = _coeff(degp, h0)
    prop = _make_prop_kernel(cpw)
    for t in range(NUM_ITER):
        p = prop(g, rowi, coli, zeros_d)
        if t < NUM_ITER - 1:
            g = _combine(p, g, h0, am, bm)
        else:
            g = _combine(p, g, h0, af, bf)
    return g
